# Initial kernel scaffold; baseline (speedup 1.0000x reference)
import jax
import jax.numpy as jnp
from jax.experimental import pallas as pl


def kernel(x, edge_index, Wl0, bl0, Wr0, Wl1, bl1, Wr1, W1, b1, W2, b2):
    raise NotImplementedError("write your pallas kernel here")


# speedup vs baseline: 1.0144x; 1.0144x over previous
import functools

import jax
import jax.numpy as jnp
from jax.experimental import pallas as pl


N = 10000
E = 320000
D = 128


def _dense_body(mean_ref, x_ref, wl_ref, bl_ref, wr_ref, out_ref):
    mean = mean_ref[...]
    x = x_ref[...]
    h = (jnp.dot(mean, wl_ref[...], preferred_element_type=jnp.float32)
         + bl_ref[...][None, :]
         + jnp.dot(x, wr_ref[...], preferred_element_type=jnp.float32))
    out_ref[...] = jnp.maximum(h, 0.0)


def _dense(mean, x, wl, bl, wr, blk=400):
    n = x.shape[0]
    grid = (n // blk,)
    return pl.pallas_call(
        _dense_body,
        grid=grid,
        in_specs=[
            pl.BlockSpec((blk, D), lambda i: (i, 0)),
            pl.BlockSpec((blk, D), lambda i: (i, 0)),
            pl.BlockSpec((D, D), lambda i: (0, 0)),
            pl.BlockSpec((D,), lambda i: (0,)),
            pl.BlockSpec((D, D), lambda i: (0, 0)),
        ],
        out_specs=pl.BlockSpec((blk, D), lambda i: (i, 0)),
        out_shape=jax.ShapeDtypeStruct((n, D), jnp.float32),
    )(mean, x, wl, bl, wr)


def _head_body(h_ref, w1_ref, b1_ref, w2_ref, b2_ref, out_ref):
    h = jnp.maximum(h_ref[...], 0.0)
    h = jnp.maximum(jnp.dot(h, w1_ref[...], preferred_element_type=jnp.float32)
                    + b1_ref[...][None, :], 0.0)
    out_ref[...] = (jnp.dot(h, w2_ref[...], preferred_element_type=jnp.float32)
                    + b2_ref[...][None, :])


def _head(h, w1, b1, w2, b2, blk=400):
    n = h.shape[0]
    return pl.pallas_call(
        _head_body,
        grid=(n // blk,),
        in_specs=[
            pl.BlockSpec((blk, D), lambda i: (i, 0)),
            pl.BlockSpec((D, D), lambda i: (0, 0)),
            pl.BlockSpec((D,), lambda i: (0,)),
            pl.BlockSpec((D, D), lambda i: (0, 0)),
            pl.BlockSpec((D,), lambda i: (0,)),
        ],
        out_specs=pl.BlockSpec((blk, D), lambda i: (i, 0)),
        out_shape=jax.ShapeDtypeStruct((n, D), jnp.float32),
    )(h, w1, b1, w2, b2)


def kernel(x, edge_index, Wl0, bl0, Wr0, Wl1, bl1, Wr1, W1, b1, W2, b2):
    src = edge_index[0]
    dst = edge_index[1]
    cnt = jax.ops.segment_sum(jnp.ones((E,), jnp.float32), dst, num_segments=N)
    inv = 1.0 / jnp.clip(cnt, 1.0, None)

    msgs = jnp.take(x, src, axis=0)
    agg = jax.ops.segment_sum(msgs, dst, num_segments=N)
    h = _dense(agg * inv[:, None], x, Wl0, bl0, Wr0)

    msgs = jnp.take(h, src, axis=0)
    agg = jax.ops.segment_sum(msgs, dst, num_segments=N)
    h = _dense(agg * inv[:, None], h, Wl1, bl1, Wr1)

    return _head(h, W1, b1, W2, b2)


# speedup vs baseline: 2.7389x; 2.7000x over previous
import functools

import jax
import jax.numpy as jnp
from jax import lax
from jax.experimental import pallas as pl
from jax.experimental.pallas import tpu as pltpu
from jax.experimental.pallas import tpu_sc as plsc

N = 10000
E = 320000
D = 128

NC = 2
NS = 16
NW = NC * NS
C = 64
NCH = 158
E_PAD = NW * NCH * C
N_T = 10112
RPS = N_T // NS
CR = N_T // D
_STRIPE_OFFS = [min(j, RPS - C) for j in range(0, RPS, C)]


def _make_sc_aggregate():
    out_type = [jax.ShapeDtypeStruct((NC, N_T, D), jnp.float32)]
    scratch = [
        pltpu.VMEM_SHARED((N_T, D), jnp.float32),
        pltpu.VMEM((C,), jnp.int32),
        pltpu.VMEM((C,), jnp.int32),
        pltpu.VMEM((C, D), jnp.float32),
        pltpu.SemaphoreType.DMA,
    ]
    def body(feat, src1, dst1, *refs):
        agg_out, agg_sh, src_v, dst_v, gbuf, sem = refs
        cid = lax.axis_index("c")
        sid = lax.axis_index("s")
        wid = cid * NS + sid

        z16 = jnp.zeros((16,), jnp.float32)

        def zrow(i, carry):
            for l in range(D // 16):
                gbuf[i, pl.ds(l * 16, 16)] = z16
            return carry

        lax.fori_loop(0, C, zrow, 0)

        base = pl.multiple_of(sid * RPS, 8)

        def fill_idx(j):
            for l in range(C // 16):
                src_v[pl.ds(l * 16, 16)] = (
                    base + j + l * 16 + lax.iota(jnp.int32, 16))

        for j in _STRIPE_OFFS:
            fill_idx(j)
            pltpu.sync_copy(gbuf, agg_sh.at[src_v])
        plsc.subcore_barrier()

        def step(ch, carry):
            off = pl.multiple_of((wid * NCH + ch) * C, 64)
            pltpu.sync_copy(src1.at[pl.ds(off, C)], src_v)
            pltpu.sync_copy(dst1.at[pl.ds(off, C)], dst_v)
            pltpu.async_copy(feat.at[src_v], gbuf, sem).wait()
            pltpu.sync_copy(gbuf, agg_sh.at[dst_v], add=True)
            return carry

        lax.fori_loop(0, NCH, step, 0)
        plsc.subcore_barrier()

        for j in _STRIPE_OFFS:
            fill_idx(j)
            pltpu.async_copy(agg_sh.at[src_v], gbuf, sem).wait()
            pltpu.sync_copy(gbuf, agg_out.at[cid, pl.ds(base + j, C)])

    mesh = plsc.VectorSubcoreMesh(core_axis_name="c", subcore_axis_name="s")
    return pl.kernel(body, out_type=tuple(out_type), mesh=mesh,
                     scratch_types=scratch)


_sc_agg = _make_sc_aggregate()


def _dense1_body(agg_ref, cnt_ref, x_ref, wl_ref, bl_ref, wr_ref, out_ref):
    scale = 1.0 / jnp.clip(cnt_ref[:, 0], 1.0, None)
    mean = (agg_ref[0] + agg_ref[1]) * scale[:, None]
    h = (jnp.dot(mean, wl_ref[...], preferred_element_type=jnp.float32)
         + bl_ref[...][None, :]
         + jnp.dot(x_ref[...], wr_ref[...], preferred_element_type=jnp.float32))
    out_ref[...] = jnp.maximum(h, 0.0)


def _dense1(agg, cnt, x, wl, bl, wr, blk=400):
    return pl.pallas_call(
        _dense1_body,
        grid=(N // blk,),
        in_specs=[
            pl.BlockSpec((NC, blk, D), lambda i: (0, i, 0)),
            pl.BlockSpec((blk, 8), lambda i: (i, 0)),
            pl.BlockSpec((blk, D), lambda i: (i, 0)),
            pl.BlockSpec((D, D), lambda i: (0, 0)),
            pl.BlockSpec((D,), lambda i: (0,)),
            pl.BlockSpec((D, D), lambda i: (0, 0)),
        ],
        out_specs=pl.BlockSpec((blk, D), lambda i: (i, 0)),
        out_shape=jax.ShapeDtypeStruct((N, D), jnp.float32),
    )(agg, cnt, x, wl, bl, wr)


def _dense2_body(agg_ref, cnt_ref, h_ref, wl_ref, bl_ref, wr_ref,
                 w1_ref, b1_ref, w2_ref, b2_ref, out_ref):
    scale = 1.0 / jnp.clip(cnt_ref[:, 0], 1.0, None)
    mean = (agg_ref[0] + agg_ref[1]) * scale[:, None]
    g = (jnp.dot(mean, wl_ref[...], preferred_element_type=jnp.float32)
         + bl_ref[...][None, :]
         + jnp.dot(h_ref[...], wr_ref[...], preferred_element_type=jnp.float32))
    g = jnp.maximum(g, 0.0)
    g = jnp.maximum(jnp.dot(g, w1_ref[...], preferred_element_type=jnp.float32)
                    + b1_ref[...][None, :], 0.0)
    out_ref[...] = (jnp.dot(g, w2_ref[...], preferred_element_type=jnp.float32)
                    + b2_ref[...][None, :])


def _dense2(agg, cnt, h, wl, bl, wr, w1, b1, w2, b2, blk=400):
    return pl.pallas_call(
        _dense2_body,
        grid=(N // blk,),
        in_specs=[
            pl.BlockSpec((NC, blk, D), lambda i: (0, i, 0)),
            pl.BlockSpec((blk, 8), lambda i: (i, 0)),
            pl.BlockSpec((blk, D), lambda i: (i, 0)),
            pl.BlockSpec((D, D), lambda i: (0, 0)),
            pl.BlockSpec((D,), lambda i: (0,)),
            pl.BlockSpec((D, D), lambda i: (0, 0)),
            pl.BlockSpec((D, D), lambda i: (0, 0)),
            pl.BlockSpec((D,), lambda i: (0,)),
            pl.BlockSpec((D, D), lambda i: (0, 0)),
            pl.BlockSpec((D,), lambda i: (0,)),
        ],
        out_specs=pl.BlockSpec((blk, D), lambda i: (i, 0)),
        out_shape=jax.ShapeDtypeStruct((N, D), jnp.float32),
    )(agg, cnt, h, wl, bl, wr, w1, b1, w2, b2)


def kernel(x, edge_index, Wl0, bl0, Wr0, Wl1, bl1, Wr1, W1, b1, W2, b2):
    pad = E_PAD - E
    src1 = jnp.concatenate([edge_index[0], jnp.zeros((pad,), jnp.int32)])
    dst1 = jnp.concatenate([edge_index[1], jnp.full((pad,), N, jnp.int32)])
    cnt = jax.ops.segment_sum(jnp.ones((E,), jnp.float32), edge_index[1],
                              num_segments=N)
    cnt = jnp.broadcast_to(cnt[:, None], (N, 8))
    (agg,) = _sc_agg(x, src1, dst1)
    h = _dense1(agg, cnt, x, Wl0, bl0, Wr0)
    (agg,) = _sc_agg(h, src1, dst1)
    return _dense2(agg, cnt, h, Wl1, bl1, Wr1, W1, b1, W2, b2)


# speedup vs baseline: 3.5640x; 1.3012x over previous
import functools

import jax
import jax.numpy as jnp
from jax import lax
from jax.experimental import pallas as pl
from jax.experimental.pallas import tpu as pltpu
from jax.experimental.pallas import tpu_sc as plsc

N = 10000
E = 320000
D = 128

NC = 2
NS = 16
NW = NC * NS
C = 64
NCH = 158
E_PAD = NW * NCH * C
N_T = 10112
RPS = N_T // NS
CR = N_T // D
_STRIPE_OFFS = [min(j, RPS - C) for j in range(0, RPS, C)]


def _make_sc_aggregate():
    out_type = [jax.ShapeDtypeStruct((NC, N_T, D), jnp.float32)]
    scratch = [
        pltpu.VMEM_SHARED((N_T, D), jnp.float32),
        pltpu.VMEM((2, C), jnp.int32),
        pltpu.VMEM((2, C), jnp.int32),
        pltpu.VMEM((C, D), jnp.float32),
        pltpu.VMEM((C, D), jnp.float32),
        pltpu.SemaphoreType.DMA,
        pltpu.SemaphoreType.DMA,
    ]

    def body(feat, sd_hbm, *refs):
        agg_out, agg_sh, sd0, sd1, gb0, gb1, sem0, sem1 = refs
        sd = (sd0, sd1)
        gb = (gb0, gb1)
        sem = (sem0, sem1)
        cid = lax.axis_index("c")
        sid = lax.axis_index("s")
        wid = cid * NS + sid

        z16 = jnp.zeros((16,), jnp.float32)

        def zrow(i, carry):
            for l in range(D // 16):
                gb0[i, pl.ds(l * 16, 16)] = z16
            return carry

        lax.fori_loop(0, C, zrow, 0)

        base = pl.multiple_of(sid * RPS, 8)

        def fill_idx(j):
            for l in range(C // 16):
                sd0[0, pl.ds(l * 16, 16)] = (
                    base + j + l * 16 + lax.iota(jnp.int32, 16))

        for j in _STRIPE_OFFS:
            fill_idx(j)
            pltpu.sync_copy(gb0, agg_sh.at[sd0.at[0]])
        plsc.subcore_barrier()

        e0 = wid * NCH
        pltpu.sync_copy(sd_hbm.at[e0], sd0)
        pltpu.async_copy(feat.at[sd0.at[0]], gb0, sem0)

        def pair(i2, carry):
            for b in (0, 1):
                k = i2 * 2 + b
                nb = 1 - b

                @pl.when(k + 1 < NCH)
                def _prefetch():
                    pltpu.sync_copy(sd_hbm.at[e0 + k + 1], sd[nb])
                    pltpu.async_copy(feat.at[sd[nb].at[0]], gb[nb], sem[nb])

                pltpu.make_async_copy(feat.at[sd[b].at[0]], gb[b],
                                      sem[b]).wait()
                pltpu.sync_copy(gb[b], agg_sh.at[sd[b].at[1]], add=True)
            return carry

        lax.fori_loop(0, NCH // 2, pair, 0)
        plsc.subcore_barrier()

        for j in _STRIPE_OFFS:
            fill_idx(j)
            pltpu.async_copy(agg_sh.at[sd0.at[0]], gb0, sem0).wait()
            pltpu.sync_copy(gb0, agg_out.at[cid, pl.ds(base + j, C)])

    mesh = plsc.VectorSubcoreMesh(core_axis_name="c", subcore_axis_name="s")
    return pl.kernel(body, out_type=tuple(out_type), mesh=mesh,
                     scratch_types=scratch)


_sc_agg = _make_sc_aggregate()


def _dense1_body(agg_ref, cnt_ref, x_ref, wl_ref, bl_ref, wr_ref, out_ref):
    scale = 1.0 / jnp.clip(cnt_ref[:, 0], 1.0, None)
    mean = (agg_ref[0] + agg_ref[1]) * scale[:, None]
    h = (jnp.dot(mean, wl_ref[...], preferred_element_type=jnp.float32)
         + bl_ref[...][None, :]
         + jnp.dot(x_ref[...], wr_ref[...], preferred_element_type=jnp.float32))
    out_ref[...] = jnp.maximum(h, 0.0)


def _dense1(agg, cnt, x, wl, bl, wr, blk=400):
    return pl.pallas_call(
        _dense1_body,
        grid=(N // blk,),
        in_specs=[
            pl.BlockSpec((NC, blk, D), lambda i: (0, i, 0)),
            pl.BlockSpec((blk, 8), lambda i: (i, 0)),
            pl.BlockSpec((blk, D), lambda i: (i, 0)),
            pl.BlockSpec((D, D), lambda i: (0, 0)),
            pl.BlockSpec((D,), lambda i: (0,)),
            pl.BlockSpec((D, D), lambda i: (0, 0)),
        ],
        out_specs=pl.BlockSpec((blk, D), lambda i: (i, 0)),
        out_shape=jax.ShapeDtypeStruct((N, D), jnp.float32),
    )(agg, cnt, x, wl, bl, wr)


def _dense2_body(agg_ref, cnt_ref, h_ref, wl_ref, bl_ref, wr_ref,
                 w1_ref, b1_ref, w2_ref, b2_ref, out_ref):
    scale = 1.0 / jnp.clip(cnt_ref[:, 0], 1.0, None)
    mean = (agg_ref[0] + agg_ref[1]) * scale[:, None]
    g = (jnp.dot(mean, wl_ref[...], preferred_element_type=jnp.float32)
         + bl_ref[...][None, :]
         + jnp.dot(h_ref[...], wr_ref[...], preferred_element_type=jnp.float32))
    g = jnp.maximum(g, 0.0)
    g = jnp.maximum(jnp.dot(g, w1_ref[...], preferred_element_type=jnp.float32)
                    + b1_ref[...][None, :], 0.0)
    out_ref[...] = (jnp.dot(g, w2_ref[...], preferred_element_type=jnp.float32)
                    + b2_ref[...][None, :])


def _dense2(agg, cnt, h, wl, bl, wr, w1, b1, w2, b2, blk=400):
    return pl.pallas_call(
        _dense2_body,
        grid=(N // blk,),
        in_specs=[
            pl.BlockSpec((NC, blk, D), lambda i: (0, i, 0)),
            pl.BlockSpec((blk, 8), lambda i: (i, 0)),
            pl.BlockSpec((blk, D), lambda i: (i, 0)),
            pl.BlockSpec((D, D), lambda i: (0, 0)),
            pl.BlockSpec((D,), lambda i: (0,)),
            pl.BlockSpec((D, D), lambda i: (0, 0)),
            pl.BlockSpec((D, D), lambda i: (0, 0)),
            pl.BlockSpec((D,), lambda i: (0,)),
            pl.BlockSpec((D, D), lambda i: (0, 0)),
            pl.BlockSpec((D,), lambda i: (0,)),
        ],
        out_specs=pl.BlockSpec((blk, D), lambda i: (i, 0)),
        out_shape=jax.ShapeDtypeStruct((N, D), jnp.float32),
    )(agg, cnt, h, wl, bl, wr, w1, b1, w2, b2)


def kernel(x, edge_index, Wl0, bl0, Wr0, Wl1, bl1, Wr1, W1, b1, W2, b2):
    pad = E_PAD - E
    src1 = jnp.concatenate([edge_index[0], jnp.zeros((pad,), jnp.int32)])
    dst1 = jnp.concatenate([edge_index[1], jnp.full((pad,), N, jnp.int32)])
    sd = jnp.stack([src1.reshape(NW * NCH, C), dst1.reshape(NW * NCH, C)],
                   axis=1)
    cnt = jax.ops.segment_sum(jnp.ones((E,), jnp.float32), edge_index[1],
                              num_segments=N)
    cnt = jnp.broadcast_to(cnt[:, None], (N, 8))
    (agg,) = _sc_agg(x, sd)
    h = _dense1(agg, cnt, x, Wl0, bl0, Wr0)
    (agg,) = _sc_agg(h, sd)
    return _dense2(agg, cnt, h, Wl1, bl1, Wr1, W1, b1, W2, b2)
